# baseline (speedup 1.0000x reference)
import functools

import jax
import jax.numpy as jnp
from jax import lax
from jax.experimental import pallas as pl
from jax.experimental.pallas import tpu as pltpu
from jax.experimental.pallas import tpu_sc as plsc

N = 10000
D = 256
D2 = 512
K = 16
EF = 4

_info = plsc.get_sparse_core_info()
NC = _info.num_cores
NS = _info.num_subcores
NW = NC * NS
TOT = EF * N
ROWS_PER_W = TOT // NW
NBUF = 5
GROUPS = ROWS_PER_W // NBUF
CH = D2 // 16
CHD = D // 16


_BN = 400


def _mm_body(q_ref, w_ref, o_ref):
    o_ref[0] = jnp.dot(q_ref[...], w_ref[0],
                       preferred_element_type=jnp.float32,
                       precision=jax.lax.Precision.HIGHEST)


def _tc_matmul(u, wt):
    return pl.pallas_call(
        _mm_body,
        grid=(EF, N // _BN),
        in_specs=[
            pl.BlockSpec((_BN, D2), lambda i, j: (j, 0)),
            pl.BlockSpec((1, D2, D2), lambda i, j: (i, 0, 0)),
        ],
        out_specs=pl.BlockSpec((1, _BN, D2), lambda i, j: (i, j, 0)),
        out_shape=jax.ShapeDtypeStruct((EF, N, D2), jnp.float32),
    )(u, wt)


def _sc_attn_body(u_hbm, wq_hbm, efs_hbm, out_hbm,
                  idxv, rowsv, wqv, outv,
                  *sems):
    wid = lax.axis_index("s") * NC + lax.axis_index("c")
    base = wid * ROWS_PER_W

    pltpu.sync_copy(efs_hbm.at[pl.ds(base * K, ROWS_PER_W * K)], idxv)

    sems_r = sems[:NBUF]
    sems_w = sems[NBUF:]

    def issue(j, b):
        pltpu.make_async_copy(u_hbm.at[idxv.at[pl.ds(j * K, K)]], rowsv.at[b],
                              sems_r[b]).start()
        pltpu.make_async_copy(wq_hbm.at[pl.ds((base + j) * D2, D2)],
                              wqv.at[b], sems_w[b]).start()

    def wait(j, b):
        pltpu.make_async_copy(u_hbm.at[idxv.at[pl.ds(j * K, K)]], rowsv.at[b],
                              sems_r[b]).wait()
        pltpu.make_async_copy(wq_hbm.at[pl.ds((base + j) * D2, D2)],
                              wqv.at[b], sems_w[b]).wait()

    def compute(b):
        zero = jnp.zeros((16,), jnp.float32)

        @plsc.parallel_loop(0, CH, 1, unroll=8, carry=(zero,) * K)
        def accs(c, acc):
            w = wqv[b, pl.ds(c * 16, 16)]
            return tuple(acc[k] + rowsv[b, k, pl.ds(c * 16, 16)] * w
                         for k in range(K))

        iota = lax.iota(jnp.int32, 16)
        parts = [jnp.where(iota == k, jnp.sum(accs[k]), 0.0)
                 for k in range(K)]
        while len(parts) > 1:
            parts = [parts[2 * i] + parts[2 * i + 1]
                     for i in range(len(parts) // 2)]
        r = parts[0]
        m = jnp.max(r)
        e = jnp.exp(r - m)
        p = e / jnp.sum(e)
        ps = [p[k] for k in range(K)]

        @plsc.parallel_loop(0, CHD, 1, unroll=2)
        def _(c):
            oacc = ps[0] * rowsv[b, 0, pl.ds(c * 16, 16)]
            for k in range(1, K):
                oacc = oacc + ps[k] * rowsv[b, k, pl.ds(c * 16, 16)]
            outv[pl.ds(b * D + c * 16, 16)] = oacc

    for b in range(NBUF):
        issue(b, b)

    def group(g, carry):
        for b in range(NBUF):
            j = g * NBUF + b
            wait(j, b)
            compute(b)
            nxt = j + NBUF

            @pl.when(nxt < ROWS_PER_W)
            def _():
                issue(nxt, b)

        pltpu.sync_copy(outv,
                        out_hbm.at[pl.ds((base + g * NBUF) * D, NBUF * D)])
        return carry

    lax.fori_loop(0, GROUPS, group, 0)


_sc_attn = functools.partial(
    pl.kernel,
    mesh=plsc.VectorSubcoreMesh(core_axis_name="c", subcore_axis_name="s"),
    compiler_params=pltpu.CompilerParams(needs_layout_passes=False,
                                         use_tc_tiling_on_sc=False),
    out_type=jax.ShapeDtypeStruct((TOT * D,), jnp.float32),
    scratch_types=[
        pltpu.VMEM((ROWS_PER_W * K,), jnp.int32),
        pltpu.VMEM((NBUF, K, D2), jnp.float32),
        pltpu.VMEM((NBUF, D2), jnp.float32),
        pltpu.VMEM((NBUF * D,), jnp.float32),
    ] + [pltpu.SemaphoreType.DMA] * (2 * NBUF),
)(_sc_attn_body)


_BC = 400


def _comb_body(v_ref, e_ref, ma_ref, mb_ref, o_ref):
    v = v_ref[...]
    ma = ma_ref[...]
    mb = mb_ref[...]
    wv = jnp.sum(v * ma, axis=1, keepdims=True)
    effs = [e_ref[i] for i in range(EF)]
    ws = []
    for i in range(EF):
        w = wv + jnp.sum(effs[i] * mb, axis=1, keepdims=True)
        ws.append(jnp.where(w >= 0, w, 0.01 * w))
    m = ws[0]
    for i in range(1, EF):
        m = jnp.maximum(m, ws[i])
    es = [jnp.exp(w - m) for w in ws]
    s = es[0]
    for i in range(1, EF):
        s = s + es[i]
    out = (es[0] / s) * jnp.maximum(effs[0], 0.0)
    for i in range(1, EF):
        out = out + (es[i] / s) * jnp.maximum(effs[i], 0.0)
    o_ref[...] = out


def _tc_combine(v_fea, eff, ma, mb):
    return pl.pallas_call(
        _comb_body,
        grid=(N // _BC,),
        in_specs=[
            pl.BlockSpec((_BC, D), lambda j: (j, 0)),
            pl.BlockSpec((EF, _BC, D), lambda j: (0, j, 0)),
            pl.BlockSpec((1, D), lambda j: (0, 0)),
            pl.BlockSpec((1, D), lambda j: (0, 0)),
        ],
        out_specs=pl.BlockSpec((_BC, D), lambda j: (j, 0)),
        out_shape=jax.ShapeDtypeStruct((N, D), jnp.float32),
    )(v_fea, eff, ma, mb)


def kernel(v_fea, t_emb, efs, Ws, mlp_w):
    u = jnp.concatenate([v_fea, t_emb], axis=-1)
    wt = jnp.swapaxes(Ws, 1, 2)
    wq = _tc_matmul(u, wt)
    efs_i = efs.reshape(TOT * K).astype(jnp.int32)
    eff = _sc_attn(u, wq.reshape(TOT * D2), efs_i)
    ma = mlp_w[:, :D]
    mb = mlp_w[:, D:]
    return _tc_combine(v_fea, eff.reshape(EF, N, D), ma, mb)


# speedup vs baseline: 1.4509x; 1.4509x over previous
import functools

import jax
import jax.numpy as jnp
from jax import lax
from jax.experimental import pallas as pl
from jax.experimental.pallas import tpu as pltpu
from jax.experimental.pallas import tpu_sc as plsc

N = 10000
D = 256
D2 = 512
K = 16
EF = 4

_info = plsc.get_sparse_core_info()
NC = _info.num_cores
NS = _info.num_subcores
NW = NC * NS
TOT = EF * N
ROWS_PER_W = TOT // NW
NBUF = 5
GROUPS = ROWS_PER_W // NBUF
CH = D2 // 16
CHD = D // 16


_BN = 400


def _mm_body(q_ref, w_ref, o_ref):
    o_ref[0] = jnp.dot(q_ref[...], w_ref[0],
                       preferred_element_type=jnp.float32,
                       precision=jax.lax.Precision.HIGHEST)


def _tc_matmul(u, wt):
    return pl.pallas_call(
        _mm_body,
        grid=(EF, N // _BN),
        in_specs=[
            pl.BlockSpec((_BN, D2), lambda i, j: (j, 0)),
            pl.BlockSpec((1, D2, D2), lambda i, j: (i, 0, 0)),
        ],
        out_specs=pl.BlockSpec((1, _BN, D2), lambda i, j: (i, j, 0)),
        out_shape=jax.ShapeDtypeStruct((EF, N, D2), jnp.float32),
    )(u, wt)


def _sc_attn_body(u_hbm, wq_hbm, efs_hbm, out_hbm,
                  idxv, rowsv, wqv, outv,
                  *sems):
    wid = lax.axis_index("s") * NC + lax.axis_index("c")
    base = wid * ROWS_PER_W

    pltpu.sync_copy(efs_hbm.at[pl.ds(base * K, ROWS_PER_W * K)], idxv)

    sems_r = sems[:NBUF]
    sems_w = sems[NBUF:]

    def issue(j, b):
        pltpu.make_async_copy(u_hbm.at[idxv.at[pl.ds(j * K, K)]], rowsv.at[b],
                              sems_r[b]).start()
        pltpu.make_async_copy(wq_hbm.at[pl.ds((base + j) * D2, D2)],
                              wqv.at[b], sems_w[b]).start()

    def wait(j, b):
        pltpu.make_async_copy(u_hbm.at[idxv.at[pl.ds(j * K, K)]], rowsv.at[b],
                              sems_r[b]).wait()
        pltpu.make_async_copy(wq_hbm.at[pl.ds((base + j) * D2, D2)],
                              wqv.at[b], sems_w[b]).wait()

    def compute(b):
        zero = jnp.zeros((16,), jnp.float32)

        @plsc.parallel_loop(0, CH, 1, unroll=2, carry=(zero,) * K)
        def accs(c, acc):
            w = wqv[b, pl.ds(c * 16, 16)]
            return tuple(acc[k] + rowsv[b, k, pl.ds(c * 16, 16)] * w
                         for k in range(K))

        iota = lax.iota(jnp.int32, 16)
        parts = [jnp.where(iota == k, jnp.sum(accs[k]), 0.0)
                 for k in range(K)]
        while len(parts) > 1:
            parts = [parts[2 * i] + parts[2 * i + 1]
                     for i in range(len(parts) // 2)]
        r = parts[0]
        m = jnp.max(r)
        e = jnp.exp(r - m)
        p = e / jnp.sum(e)
        ps = [p[k] for k in range(K)]

        @plsc.parallel_loop(0, CHD, 1, unroll=2)
        def _(c):
            oacc = ps[0] * rowsv[b, 0, pl.ds(c * 16, 16)]
            for k in range(1, K):
                oacc = oacc + ps[k] * rowsv[b, k, pl.ds(c * 16, 16)]
            outv[pl.ds(b * D + c * 16, 16)] = oacc

    for b in range(NBUF):
        issue(b, b)

    def group(g, carry):
        for b in range(NBUF):
            j = g * NBUF + b
            wait(j, b)
            compute(b)
            nxt = j + NBUF

            @pl.when(nxt < ROWS_PER_W)
            def _():
                issue(nxt, b)

        pltpu.sync_copy(outv,
                        out_hbm.at[pl.ds((base + g * NBUF) * D, NBUF * D)])
        return carry

    lax.fori_loop(0, GROUPS, group, 0)


_sc_attn = functools.partial(
    pl.kernel,
    mesh=plsc.VectorSubcoreMesh(core_axis_name="c", subcore_axis_name="s"),
    compiler_params=pltpu.CompilerParams(needs_layout_passes=False,
                                         use_tc_tiling_on_sc=False),
    out_type=jax.ShapeDtypeStruct((TOT * D,), jnp.float32),
    scratch_types=[
        pltpu.VMEM((ROWS_PER_W * K,), jnp.int32),
        pltpu.VMEM((NBUF, K, D2), jnp.float32),
        pltpu.VMEM((NBUF, D2), jnp.float32),
        pltpu.VMEM((NBUF * D,), jnp.float32),
    ] + [pltpu.SemaphoreType.DMA] * (2 * NBUF),
)(_sc_attn_body)


_BC = 400


def _comb_body(v_ref, e_ref, ma_ref, mb_ref, o_ref):
    v = v_ref[...]
    ma = ma_ref[...]
    mb = mb_ref[...]
    wv = jnp.sum(v * ma, axis=1, keepdims=True)
    effs = [e_ref[i] for i in range(EF)]
    ws = []
    for i in range(EF):
        w = wv + jnp.sum(effs[i] * mb, axis=1, keepdims=True)
        ws.append(jnp.where(w >= 0, w, 0.01 * w))
    m = ws[0]
    for i in range(1, EF):
        m = jnp.maximum(m, ws[i])
    es = [jnp.exp(w - m) for w in ws]
    s = es[0]
    for i in range(1, EF):
        s = s + es[i]
    out = (es[0] / s) * jnp.maximum(effs[0], 0.0)
    for i in range(1, EF):
        out = out + (es[i] / s) * jnp.maximum(effs[i], 0.0)
    o_ref[...] = out


def _tc_combine(v_fea, eff, ma, mb):
    return pl.pallas_call(
        _comb_body,
        grid=(N // _BC,),
        in_specs=[
            pl.BlockSpec((_BC, D), lambda j: (j, 0)),
            pl.BlockSpec((EF, _BC, D), lambda j: (0, j, 0)),
            pl.BlockSpec((1, D), lambda j: (0, 0)),
            pl.BlockSpec((1, D), lambda j: (0, 0)),
        ],
        out_specs=pl.BlockSpec((_BC, D), lambda j: (j, 0)),
        out_shape=jax.ShapeDtypeStruct((N, D), jnp.float32),
    )(v_fea, eff, ma, mb)


def kernel(v_fea, t_emb, efs, Ws, mlp_w):
    u = jnp.concatenate([v_fea, t_emb], axis=-1)
    wt = jnp.swapaxes(Ws, 1, 2)
    wq = _tc_matmul(u, wt)
    efs_i = efs.reshape(TOT * K).astype(jnp.int32)
    eff = _sc_attn(u, wq.reshape(TOT * D2), efs_i)
    ma = mlp_w[:, :D]
    mb = mlp_w[:, D:]
    return _tc_combine(v_fea, eff.reshape(EF, N, D), ma, mb)


# speedup vs baseline: 1.4623x; 1.0078x over previous
import functools

import jax
import jax.numpy as jnp
from jax import lax
from jax.experimental import pallas as pl
from jax.experimental.pallas import tpu as pltpu
from jax.experimental.pallas import tpu_sc as plsc

N = 10000
D = 256
D2 = 512
K = 16
EF = 4

_info = plsc.get_sparse_core_info()
NC = _info.num_cores
NS = _info.num_subcores
NW = NC * NS
TOT = EF * N
ROWS_PER_W = TOT // NW
NBUF = 5
GROUPS = ROWS_PER_W // NBUF
CH = D2 // 16
CHD = D // 16


_BN = 400


def _mm_body(v_ref, t_ref, w_ref, o_ref):
    o_ref[0] = (jnp.dot(v_ref[...], w_ref[0, :D],
                        preferred_element_type=jnp.float32,
                        precision=jax.lax.Precision.HIGHEST)
                + jnp.dot(t_ref[...], w_ref[0, D:],
                          preferred_element_type=jnp.float32,
                          precision=jax.lax.Precision.HIGHEST))


def _tc_matmul(v_fea, t_emb, wt):
    return pl.pallas_call(
        _mm_body,
        grid=(EF, N // _BN),
        in_specs=[
            pl.BlockSpec((_BN, D), lambda i, j: (j, 0)),
            pl.BlockSpec((_BN, D), lambda i, j: (j, 0)),
            pl.BlockSpec((1, D2, D2), lambda i, j: (i, 0, 0)),
        ],
        out_specs=pl.BlockSpec((1, _BN, D2), lambda i, j: (i, j, 0)),
        out_shape=jax.ShapeDtypeStruct((EF, N, D2), jnp.float32),
    )(v_fea, t_emb, wt)


def _sc_attn_body(v_hbm, t_hbm, wq_hbm, efs_hbm, out_hbm,
                  idxv, rowsv, wqv, outv,
                  *sems):
    wid = lax.axis_index("s") * NC + lax.axis_index("c")
    base = wid * ROWS_PER_W

    pltpu.sync_copy(efs_hbm.at[pl.ds(base * K, ROWS_PER_W * K)], idxv)

    sems_v = sems[:NBUF]
    sems_t = sems[NBUF:2 * NBUF]
    sems_w = sems[2 * NBUF:]

    def issue(j, b):
        pltpu.make_async_copy(v_hbm.at[idxv.at[pl.ds(j * K, K)]],
                              rowsv.at[b, 0], sems_v[b]).start()
        pltpu.make_async_copy(t_hbm.at[idxv.at[pl.ds(j * K, K)]],
                              rowsv.at[b, 1], sems_t[b]).start()
        pltpu.make_async_copy(wq_hbm.at[pl.ds((base + j) * D2, D2)],
                              wqv.at[b], sems_w[b]).start()

    def wait(j, b):
        pltpu.make_async_copy(v_hbm.at[idxv.at[pl.ds(j * K, K)]],
                              rowsv.at[b, 0], sems_v[b]).wait()
        pltpu.make_async_copy(t_hbm.at[idxv.at[pl.ds(j * K, K)]],
                              rowsv.at[b, 1], sems_t[b]).wait()
        pltpu.make_async_copy(wq_hbm.at[pl.ds((base + j) * D2, D2)],
                              wqv.at[b], sems_w[b]).wait()

    def compute(b):
        zero = jnp.zeros((16,), jnp.float32)

        @plsc.parallel_loop(0, CHD, 1, unroll=1, carry=(zero,) * K)
        def accs(c, acc):
            wv = wqv[b, pl.ds(c * 16, 16)]
            wt = wqv[b, pl.ds(D + c * 16, 16)]
            return tuple(acc[k]
                         + rowsv[b, 0, k, pl.ds(c * 16, 16)] * wv
                         + rowsv[b, 1, k, pl.ds(c * 16, 16)] * wt
                         for k in range(K))

        iota = lax.iota(jnp.int32, 16)
        parts = [jnp.where(iota == k, jnp.sum(accs[k]), 0.0)
                 for k in range(K)]
        while len(parts) > 1:
            parts = [parts[2 * i] + parts[2 * i + 1]
                     for i in range(len(parts) // 2)]
        r = parts[0]
        m = jnp.max(r)
        e = jnp.exp(r - m)
        p = e / jnp.sum(e)
        ps = [p[k] for k in range(K)]

        @plsc.parallel_loop(0, CHD, 1, unroll=2)
        def _(c):
            oacc = ps[0] * rowsv[b, 0, 0, pl.ds(c * 16, 16)]
            for k in range(1, K):
                oacc = oacc + ps[k] * rowsv[b, 0, k, pl.ds(c * 16, 16)]
            outv[pl.ds(b * D + c * 16, 16)] = oacc

    for b in range(NBUF):
        issue(b, b)

    def group(g, carry):
        for b in range(NBUF):
            j = g * NBUF + b
            wait(j, b)
            compute(b)
            nxt = j + NBUF

            @pl.when(nxt < ROWS_PER_W)
            def _():
                issue(nxt, b)

        pltpu.sync_copy(outv,
                        out_hbm.at[pl.ds((base + g * NBUF) * D, NBUF * D)])
        return carry

    lax.fori_loop(0, GROUPS, group, 0)


_sc_attn = functools.partial(
    pl.kernel,
    mesh=plsc.VectorSubcoreMesh(core_axis_name="c", subcore_axis_name="s"),
    compiler_params=pltpu.CompilerParams(needs_layout_passes=False,
                                         use_tc_tiling_on_sc=False),
    out_type=jax.ShapeDtypeStruct((TOT * D,), jnp.float32),
    scratch_types=[
        pltpu.VMEM((ROWS_PER_W * K,), jnp.int32),
        pltpu.VMEM((NBUF, 2, K, D), jnp.float32),
        pltpu.VMEM((NBUF, D2), jnp.float32),
        pltpu.VMEM((NBUF * D,), jnp.float32),
    ] + [pltpu.SemaphoreType.DMA] * (3 * NBUF),
)(_sc_attn_body)


_BC = 400


def _comb_body(v_ref, e_ref, ma_ref, mb_ref, o_ref):
    v = v_ref[...]
    ma = ma_ref[...]
    mb = mb_ref[...]
    wv = jnp.sum(v * ma, axis=1, keepdims=True)
    effs = [e_ref[i] for i in range(EF)]
    ws = []
    for i in range(EF):
        w = wv + jnp.sum(effs[i] * mb, axis=1, keepdims=True)
        ws.append(jnp.where(w >= 0, w, 0.01 * w))
    m = ws[0]
    for i in range(1, EF):
        m = jnp.maximum(m, ws[i])
    es = [jnp.exp(w - m) for w in ws]
    s = es[0]
    for i in range(1, EF):
        s = s + es[i]
    out = (es[0] / s) * jnp.maximum(effs[0], 0.0)
    for i in range(1, EF):
        out = out + (es[i] / s) * jnp.maximum(effs[i], 0.0)
    o_ref[...] = out


def _tc_combine(v_fea, eff, ma, mb):
    return pl.pallas_call(
        _comb_body,
        grid=(N // _BC,),
        in_specs=[
            pl.BlockSpec((_BC, D), lambda j: (j, 0)),
            pl.BlockSpec((EF, _BC, D), lambda j: (0, j, 0)),
            pl.BlockSpec((1, D), lambda j: (0, 0)),
            pl.BlockSpec((1, D), lambda j: (0, 0)),
        ],
        out_specs=pl.BlockSpec((_BC, D), lambda j: (j, 0)),
        out_shape=jax.ShapeDtypeStruct((N, D), jnp.float32),
    )(v_fea, eff, ma, mb)


def kernel(v_fea, t_emb, efs, Ws, mlp_w):
    wt = jnp.swapaxes(Ws, 1, 2)
    wq = _tc_matmul(v_fea, t_emb, wt)
    efs_i = efs.reshape(TOT * K).astype(jnp.int32)
    eff = _sc_attn(v_fea, t_emb, wq.reshape(TOT * D2), efs_i)
    ma = mlp_w[:, :D]
    mb = mlp_w[:, D:]
    return _tc_combine(v_fea, eff.reshape(EF, N, D), ma, mb)
